# baseline (speedup 1.0000x reference)
import functools

import jax
import jax.numpy as jnp
from jax import lax
from jax.experimental import pallas as pl
from jax.experimental.pallas import tpu as pltpu
from jax.experimental.pallas import tpu_sc as plsc

NC = 2
NS = 16
NW = NC * NS

BATCH = 1024
SEQ = 200
EMBED = 128
LANES = 16
SPW = BATCH // NW
U = 80
UPW = SPW * SEQ // U
NBUF = 6
D = 4


def _emb_body(idx_hbm, tok_hbm, pos_hbm, out_hbm, pos_v, idx_v, *rest):
    bufs = rest[:NBUF]
    gsems = rest[NBUF:2 * NBUF]
    wsems = rest[2 * NBUF:3 * NBUF]

    wid = lax.axis_index("s") * NC + lax.axis_index("c")

    pltpu.sync_copy(pos_hbm.at[pl.ds(0, SEQ)], pos_v)
    pltpu.sync_copy(idx_hbm.at[pl.ds(wid * UPW, UPW)], idx_v)

    def fire_gather(u, r):
        pltpu.async_copy(tok_hbm.at[idx_v.at[u]], bufs[r], gsems[r])

    def wait_gather(r):
        pltpu.make_async_copy(tok_hbm.at[pl.ds(0, U)], bufs[r], gsems[r]).wait()

    def fire_writeout(u, r):
        base = wid * SPW * SEQ + u * U
        pltpu.async_copy(bufs[r], out_hbm.at[pl.ds(base, U)], wsems[r])

    def wait_writeout(r):
        pltpu.make_async_copy(bufs[r], out_hbm.at[pl.ds(0, U)], wsems[r]).wait()

    def add_pos(u, r):
        p0 = lax.rem(u * U, SEQ)
        n1 = lax.min(SEQ - p0, U)
        buf = bufs[r]

        def body1(i, c):
            for j in range(EMBED // LANES):
                sl = pl.ds(j * LANES, LANES)
                buf[i, sl] = buf[i, sl] + pos_v[p0 + i, sl]
            return c
        lax.fori_loop(0, n1, body1, 0)

        def body2(i, c):
            for j in range(EMBED // LANES):
                sl = pl.ds(j * LANES, LANES)
                buf[i, sl] = buf[i, sl] + pos_v[p0 - SEQ + i, sl]
            return c
        lax.fori_loop(n1, U, body2, 0)

    def stage(u, r, wait_prev=True, fire_next=True):
        if fire_next:
            nr = (r + D) % NBUF
            if wait_prev:
                wait_writeout(nr)
            fire_gather(u + D, nr)
        wait_gather(r)
        add_pos(u, r)
        fire_writeout(u, r)

    for u in range(D):
        fire_gather(u, u % NBUF)
    A = NBUF - D
    for u in range(A):
        stage(u, u % NBUF, wait_prev=False)

    nit = (UPW - D - A) // NBUF
    tail = (UPW - D - A) - nit * NBUF

    def ring_body(g, c):
        u0 = A + g * NBUF
        for k in range(NBUF):
            stage(u0 + k, (A + k) % NBUF)
        return c

    lax.fori_loop(0, nit, ring_body, 0)

    for k in range(tail):
        u = A + nit * NBUF + k
        stage(u, u % NBUF)
    for u in range(UPW - D, UPW):
        stage(u, u % NBUF, fire_next=False)

    for r in range(NBUF):
        wait_writeout(r)


def kernel(input, tok_table, pos_table):
    idx = input.astype(jnp.int32).reshape(BATCH * SEQ // U, U)

    mesh = plsc.VectorSubcoreMesh(
        core_axis_name="c", subcore_axis_name="s", num_cores=NC, num_subcores=NS
    )
    emb = functools.partial(
        pl.kernel,
        out_type=jax.ShapeDtypeStruct((BATCH * SEQ, EMBED), jnp.float32),
        mesh=mesh,
        scratch_types=(
            [
                pltpu.VMEM((SEQ, EMBED), jnp.float32),
                pltpu.VMEM((UPW, U), jnp.int32),
            ]
            + [pltpu.VMEM((U, EMBED), jnp.float32) for _ in range(NBUF)]
            + [pltpu.SemaphoreType.DMA for _ in range(2 * NBUF)]
        ),
    )(_emb_body)
    out = emb(idx, tok_table, pos_table)
    return out.reshape(BATCH, SEQ, EMBED)


# speedup vs baseline: 2.7316x; 2.7316x over previous
import functools

import jax
import jax.numpy as jnp
from jax import lax
from jax.experimental import pallas as pl
from jax.experimental.pallas import tpu as pltpu
from jax.experimental.pallas import tpu_sc as plsc

NC = 2
NS = 16
NW = NC * NS

BATCH = 1024
SEQ = 200
EMBED = 128
LANES = 16
SPW = BATCH // NW
U = 80
UPW = SPW * SEQ // U
NBUF = 5
D = 3


def _emb_body(idx_hbm, tok_hbm, pos_hbm, out_hbm, pos_v, idx_v, *rest):
    bufs = rest[:NBUF]
    gsems = rest[NBUF:2 * NBUF]
    wsems = rest[2 * NBUF:3 * NBUF]

    wid = lax.axis_index("s") * NC + lax.axis_index("c")

    pltpu.sync_copy(pos_hbm.at[pl.ds(0, SEQ)], pos_v)
    pltpu.sync_copy(idx_hbm.at[pl.ds(wid * UPW, UPW)], idx_v)

    def fire_gather(u, r):
        pltpu.async_copy(tok_hbm.at[idx_v.at[u]], bufs[r], gsems[r])

    def wait_gather(r):
        pltpu.make_async_copy(tok_hbm.at[pl.ds(0, U)], bufs[r], gsems[r]).wait()

    def fire_writeout(u, r):
        base = wid * SPW * SEQ + u * U
        pltpu.async_copy(bufs[r], out_hbm.at[pl.ds(base, U)], wsems[r])

    def wait_writeout(r):
        pltpu.make_async_copy(bufs[r], out_hbm.at[pl.ds(0, U)], wsems[r]).wait()

    def add_pos(u, r):
        del u
        p0 = (r * U) % SEQ
        n1 = min(SEQ - p0, U)
        buf = bufs[r]

        def body1(i, c):
            for j in range(EMBED // LANES):
                sl = pl.ds(j * LANES, LANES)
                buf[i, sl] = buf[i, sl] + pos_v[p0 + i, sl]
            return c
        lax.fori_loop(0, n1, body1, 0)

        if n1 < U:
            def body2(i, c):
                for j in range(EMBED // LANES):
                    sl = pl.ds(j * LANES, LANES)
                    buf[i, sl] = buf[i, sl] + pos_v[p0 - SEQ + i, sl]
                return c
            lax.fori_loop(n1, U, body2, 0)

    def stage(u, r, wait_prev=True, fire_next=True):
        if fire_next:
            nr = (r + D) % NBUF
            if wait_prev:
                wait_writeout(nr)
            fire_gather(u + D, nr)
        wait_gather(r)
        add_pos(u, r)
        fire_writeout(u, r)

    for u in range(D):
        fire_gather(u, u % NBUF)
    A = NBUF - D
    for u in range(A):
        stage(u, u % NBUF, wait_prev=False)

    nit = (UPW - D - A) // NBUF
    tail = (UPW - D - A) - nit * NBUF

    def ring_body(g, c):
        u0 = A + g * NBUF
        for k in range(NBUF):
            stage(u0 + k, (A + k) % NBUF)
        return c

    lax.fori_loop(0, nit, ring_body, 0)

    for k in range(tail):
        u = A + nit * NBUF + k
        stage(u, u % NBUF)
    for u in range(UPW - D, UPW):
        stage(u, u % NBUF, fire_next=False)

    for r in range(NBUF):
        wait_writeout(r)


def kernel(input, tok_table, pos_table):
    idx = input.astype(jnp.int32).reshape(BATCH * SEQ // U, U)

    mesh = plsc.VectorSubcoreMesh(
        core_axis_name="c", subcore_axis_name="s", num_cores=NC, num_subcores=NS
    )
    emb = functools.partial(
        pl.kernel,
        out_type=jax.ShapeDtypeStruct((BATCH * SEQ, EMBED), jnp.float32),
        mesh=mesh,
        scratch_types=(
            [
                pltpu.VMEM((SEQ, EMBED), jnp.float32),
                pltpu.VMEM((UPW, U), jnp.int32),
            ]
            + [pltpu.VMEM((U, EMBED), jnp.float32) for _ in range(NBUF)]
            + [pltpu.SemaphoreType.DMA for _ in range(2 * NBUF)]
        ),
    )(_emb_body)
    out = emb(idx, tok_table, pos_table)
    return out.reshape(BATCH, SEQ, EMBED)
